# baseline (speedup 1.0000x reference)
import functools

import jax
import jax.numpy as jnp
from jax import lax
from jax.experimental import pallas as pl
from jax.experimental.pallas import tpu as pltpu
from jax.experimental.pallas import tpu_sc as plsc

_SC_INFO = plsc.get_sparse_core_info()
_NC = _SC_INFO.num_cores
_NS = _SC_INFO.num_subcores
_NW = _NC * _NS


_SEQ, _D = 8192, 2048
_ROWS_PER_W = _SEQ // _NW
_C = 16
_CHUNKS = []
_off = 0
while _off < _ROWS_PER_W:
    _sz = min(_C, _ROWS_PER_W - _off)
    _CHUNKS.append((_off, _sz))
    _off += _sz
_NCH = len(_CHUNKS)
_NBUF = 3
_LEAD = 1


def _sc_copy_body(enc_hbm, out_hbm, *scratch):
    bufs = scratch[:_NBUF]
    gsems = scratch[_NBUF:2 * _NBUF]
    ssems = scratch[2 * _NBUF:3 * _NBUF]
    wid = lax.axis_index("s") * _NC + lax.axis_index("c")
    base = wid * _ROWS_PER_W

    def start_gather(g):
        off, sz = _CHUNKS[g]
        return pltpu.async_copy(
            enc_hbm.at[pl.ds(base + off, sz)],
            bufs[g % _NBUF].at[pl.ds(0, sz)],
            gsems[g % _NBUF],
        )

    def start_scatter(g):
        off, sz = _CHUNKS[g]
        return pltpu.async_copy(
            bufs[g % _NBUF].at[pl.ds(0, sz)],
            out_hbm.at[pl.ds(base + off, sz)],
            ssems[g % _NBUF],
        )

    gat = [None] * _NCH
    scat = [None] * _NCH
    for j in range(_LEAD):
        gat[j] = start_gather(j)
    for g in range(_NCH):
        j = g + _LEAD
        if j < _NCH:
            jn = j - _NBUF
            if jn >= 0:
                scat[jn].wait()
            gat[j] = start_gather(j)
        gat[g].wait()
        scat[g] = start_scatter(g)
    for g in range(max(0, _NCH - _NBUF), _NCH):
        scat[g].wait()


def kernel(x, encodings):
    seq, d = encodings.shape
    mesh = plsc.VectorSubcoreMesh(core_axis_name="c", subcore_axis_name="s")
    out = pl.kernel(
        _sc_copy_body,
        mesh=mesh,
        out_type=jax.ShapeDtypeStruct((seq, d), jnp.float32),
        scratch_types=(
            [pltpu.VMEM((_C, _D), jnp.float32)] * _NBUF
            + [pltpu.SemaphoreType.DMA] * (2 * _NBUF)
        ),
    )(encodings)
    return out[None, :, :]
